# baseline (speedup 1.0000x reference)
import functools

import jax
import jax.numpy as jnp
from jax import lax
from jax.experimental import pallas as pl
from jax.experimental.pallas import tpu as pltpu
from jax.experimental.pallas import tpu_sc as plsc

N_USERS = 5000
N_ITEMS = 5000
D = 256
E = 160000

NC = 2
NS = 16
LANES = 16

NPAD = 5120
W = D + LANES

EDGES_PER_TILE = E // NS
ECHUNK = 40
NSTEPS = EDGES_PER_TILE // ECHUNK
ROWS_PER_TILE = NPAD // NS
DCHUNK = 40

SCORE_EPT = E // NS
SCHUNK = 2000

@functools.lru_cache(maxsize=None)
def _mesh():
    return plsc.VectorSubcoreMesh(
        core_axis_name="c", subcore_axis_name="s", num_cores=NC, num_subcores=NS
    )


def _layer_body(gidx0, sidx0, tab0, gidx1, sidx1, tab1, zrows,
                out0, out1, rows_a, rows_b, gidx_v, sidx_v, acc,
                sem_a, sem_b):
    cid = lax.axis_index("c")
    sid = lax.axis_index("s")

    pltpu.sync_copy(zrows, acc.at[pl.ds(sid * ROWS_PER_TILE, ROWS_PER_TILE)])
    plsc.subcore_barrier()

    def run(gidx_hbm, sidx_hbm, tab_hbm, out_hbm):
        ebase = sid * EDGES_PER_TILE
        pltpu.sync_copy(gidx_hbm.at[pl.ds(ebase, EDGES_PER_TILE)], gidx_v)
        pltpu.sync_copy(sidx_hbm.at[pl.ds(ebase, EDGES_PER_TILE)], sidx_v)

        def start(s, buf, sem):
            return pltpu.async_copy(
                tab_hbm.at[gidx_v.at[pl.ds(s * ECHUNK, ECHUNK)]], buf, sem)

        def drain(s, buf, sem):
            pltpu.make_async_copy(
                tab_hbm.at[gidx_v.at[pl.ds(0, ECHUNK)]], buf, sem).wait()
            pltpu.sync_copy(
                buf, acc.at[sidx_v.at[pl.ds(s * ECHUNK, ECHUNK)]], add=True)

        start(0, rows_a, sem_a)
        start(1, rows_b, sem_b)

        def edge_step(g, carry):
            s = g * 2
            drain(s, rows_a, sem_a)

            @pl.when(s + 2 < NSTEPS)
            def _():
                start(s + 2, rows_a, sem_a)

            drain(s + 1, rows_b, sem_b)

            @pl.when(s + 3 < NSTEPS)
            def _():
                start(s + 3, rows_b, sem_b)

            return carry

        lax.fori_loop(0, NSTEPS // 2, edge_step, 0)
        plsc.subcore_barrier()

        lane = lax.iota(jnp.int32, LANES)
        e0 = jnp.where(lane == 0, 1.0, 0.0).astype(jnp.float32)

        def div_chunk(ci, carry):
            r0 = sid * ROWS_PER_TILE + ci * DCHUNK
            pltpu.sync_copy(acc.at[pl.ds(r0, DCHUNK)], rows_a)

            def row_step(r, c2):
                degv = rows_a[r, pl.ds(D, LANES)]
                dsplat = jnp.full((LANES,), degv[0], jnp.float32)
                scale = 1.0 / jnp.maximum(dsplat, 1.0)
                for j in range(D // LANES):
                    rows_a[r, pl.ds(j * LANES, LANES)] = (
                        rows_a[r, pl.ds(j * LANES, LANES)] * scale)
                rows_a[r, pl.ds(D, LANES)] = e0
                return c2

            lax.fori_loop(0, DCHUNK, row_step, 0)
            pltpu.sync_copy(rows_a, out_hbm.at[pl.ds(r0, DCHUNK)])
            return carry

        lax.fori_loop(0, ROWS_PER_TILE // DCHUNK, div_chunk, 0)

    @pl.when(cid == 0)
    def _():
        run(gidx0, sidx0, tab0, out0)

    @pl.when(cid == 1)
    def _():
        run(gidx1, sidx1, tab1, out1)


@functools.lru_cache(maxsize=None)
def _layer_call():
    return pl.kernel(
        _layer_body,
        out_type=(
            jax.ShapeDtypeStruct((NPAD, W), jnp.float32),
            jax.ShapeDtypeStruct((NPAD, W), jnp.float32),
        ),
        mesh=_mesh(),
        compiler_params=pltpu.CompilerParams(use_tc_tiling_on_sc=False),
        scratch_types=[
            pltpu.VMEM((ECHUNK, W), jnp.float32),
            pltpu.VMEM((ECHUNK, W), jnp.float32),
            pltpu.VMEM((EDGES_PER_TILE,), jnp.int32),
            pltpu.VMEM((EDGES_PER_TILE,), jnp.int32),
            pltpu.VMEM_SHARED((NPAD, W), jnp.float32),
            pltpu.SemaphoreType.DMA,
            pltpu.SemaphoreType.DMA,
        ],
    )


def _layer(tu, ti, user_tab, item_tab, zrows):
    new_item, new_user = _layer_call()(tu, ti, user_tab, ti, tu, item_tab, zrows)
    return new_item, new_user


def _score_body(s16, pu, pi, nu, ni, out_pos, out_neg,
                uv, iv, fv, lv, rows_v, out_v, sem):
    cid = lax.axis_index("c")
    sid = lax.axis_index("s")

    def run(uidx, iidx, out_hbm):
        def step(it, carry):
            off = sid * SCORE_EPT + it * SCHUNK
            pltpu.sync_copy(uidx.at[pl.ds(off, SCHUNK)], uv)
            pltpu.sync_copy(iidx.at[pl.ds(off, SCHUNK)], iv)

            def flat_step(j, c2):
                u = uv[pl.ds(j * LANES, LANES)]
                i = iv[pl.ds(j * LANES, LANES)]
                fv[pl.ds(j * LANES, LANES)] = u * (NPAD // LANES) + (
                    i >> 4)
                lv[pl.ds(j * LANES, LANES)] = i & (LANES - 1)
                return c2

            lax.fori_loop(0, SCHUNK // LANES, flat_step, 0)
            pltpu.async_copy(s16.at[fv], rows_v, sem).wait()
            lane = lax.iota(jnp.int32, LANES)

            def ext_step(j, c2):
                rows_idx = lane + j * LANES
                lanes_idx = lv[pl.ds(j * LANES, LANES)]
                out_v[pl.ds(j * LANES, LANES)] = plsc.load_gather(
                    rows_v, [rows_idx, lanes_idx])
                return c2

            lax.fori_loop(0, SCHUNK // LANES, ext_step, 0)
            pltpu.sync_copy(out_v, out_hbm.at[pl.ds(off, SCHUNK)])
            return carry

        lax.fori_loop(0, SCORE_EPT // SCHUNK, step, 0)

    @pl.when(cid == 0)
    def _():
        run(pu, pi, out_pos)

    @pl.when(cid == 1)
    def _():
        run(nu, ni, out_neg)


@functools.lru_cache(maxsize=None)
def _score_call():
    return pl.kernel(
        _score_body,
        out_type=(
            jax.ShapeDtypeStruct((E,), jnp.float32),
            jax.ShapeDtypeStruct((E,), jnp.float32),
        ),
        mesh=_mesh(),
        compiler_params=pltpu.CompilerParams(
            use_tc_tiling_on_sc=False, needs_layout_passes=False),
        scratch_types=[
            pltpu.VMEM((SCHUNK,), jnp.int32),
            pltpu.VMEM((SCHUNK,), jnp.int32),
            pltpu.VMEM((SCHUNK,), jnp.int32),
            pltpu.VMEM((SCHUNK,), jnp.int32),
            pltpu.VMEM((SCHUNK, LANES), jnp.float32),
            pltpu.VMEM((SCHUNK,), jnp.float32),
            pltpu.SemaphoreType.DMA,
        ],
    )


def _mm_body(x_ref, y_ref, o_ref):
    o_ref[...] = lax.dot_general(
        x_ref[...], y_ref[...],
        dimension_numbers=(((1,), (0,)), ((), ())),
        preferred_element_type=jnp.float32,
    )


def _score_matrix(us2, it2t):
    BM = BN = 256
    return pl.pallas_call(
        _mm_body,
        grid=(NPAD // BM, NPAD // BN),
        in_specs=[
            pl.BlockSpec((BM, D), lambda i, j: (i, 0)),
            pl.BlockSpec((D, BN), lambda i, j: (0, j)),
        ],
        out_specs=pl.BlockSpec((BM, BN), lambda i, j: (i, j)),
        out_shape=jax.ShapeDtypeStruct((NPAD, NPAD), jnp.float32),
    )(us2, it2t)


def _augment(emb):
    tab = jnp.zeros((NPAD, W), jnp.float32)
    tab = tab.at[:emb.shape[0], :D].set(emb)
    tab = tab.at[:, D].set(1.0)
    return tab


def kernel(train_edge_index, pos_edge_index, neg_edge_index,
           user_embedding, item_embedding):
    tu = train_edge_index[0]
    ti = train_edge_index[1]
    user_tab = _augment(user_embedding)
    item_tab = _augment(item_embedding)
    zrows = jnp.zeros((ROWS_PER_TILE, W), jnp.float32)

    it1, us1 = _layer(tu, ti, user_tab, item_tab, zrows)
    it2, us2 = _layer(tu, ti, us1, it1, zrows)

    s = _score_matrix(us2, jnp.transpose(it2[:, :D]))
    score_pos, score_neg = _score_call()(
        s.reshape(NPAD * NPAD // LANES, LANES),
        pos_edge_index[0], pos_edge_index[1],
        neg_edge_index[0], neg_edge_index[1],
    )
    return (score_pos.reshape(E, 1), score_neg.reshape(E, 1))


# speedup vs baseline: 1.3108x; 1.3108x over previous
import functools

import jax
import jax.numpy as jnp
from jax import lax
from jax.experimental import pallas as pl
from jax.experimental.pallas import tpu as pltpu
from jax.experimental.pallas import tpu_sc as plsc

N_USERS = 5000
N_ITEMS = 5000
D = 256
E = 160000

NC = 2
NS = 16
LANES = 16

NPAD = 5120
W = D + LANES

EDGES_PER_TILE = E // NS
ECHUNK = 40
NSTEPS = EDGES_PER_TILE // ECHUNK
ROWS_PER_TILE = NPAD // NS
DCHUNK = 40

SCORE_EPT = E // NS
SCHUNK = 2000

@functools.lru_cache(maxsize=None)
def _mesh():
    return plsc.VectorSubcoreMesh(
        core_axis_name="c", subcore_axis_name="s", num_cores=NC, num_subcores=NS
    )


def _layer_body(gidx0, sidx0, tab0, gidx1, sidx1, tab1, zrows,
                out0, out1, rows_a, rows_b, gidx_v, sidx_v, acc,
                sem_a, sem_b):
    cid = lax.axis_index("c")
    sid = lax.axis_index("s")

    pltpu.sync_copy(zrows, acc.at[pl.ds(sid * ROWS_PER_TILE, ROWS_PER_TILE)])
    plsc.subcore_barrier()

    def run(gidx_hbm, sidx_hbm, tab_hbm, out_hbm):
        ebase = sid * EDGES_PER_TILE
        pltpu.sync_copy(gidx_hbm.at[pl.ds(ebase, EDGES_PER_TILE)], gidx_v)
        pltpu.sync_copy(sidx_hbm.at[pl.ds(ebase, EDGES_PER_TILE)], sidx_v)

        def start(s, buf, sem):
            return pltpu.async_copy(
                tab_hbm.at[gidx_v.at[pl.ds(s * ECHUNK, ECHUNK)]], buf, sem)

        def drain(s, buf, sem):
            pltpu.make_async_copy(
                tab_hbm.at[gidx_v.at[pl.ds(0, ECHUNK)]], buf, sem).wait()
            pltpu.sync_copy(
                buf, acc.at[sidx_v.at[pl.ds(s * ECHUNK, ECHUNK)]], add=True)

        start(0, rows_a, sem_a)
        start(1, rows_b, sem_b)

        def edge_step(g, carry):
            s = g * 2
            drain(s, rows_a, sem_a)

            @pl.when(s + 2 < NSTEPS)
            def _():
                start(s + 2, rows_a, sem_a)

            drain(s + 1, rows_b, sem_b)

            @pl.when(s + 3 < NSTEPS)
            def _():
                start(s + 3, rows_b, sem_b)

            return carry

        lax.fori_loop(0, NSTEPS // 2, edge_step, 0)
        plsc.subcore_barrier()

        lane = lax.iota(jnp.int32, LANES)
        e0 = jnp.where(lane == 0, 1.0, 0.0).astype(jnp.float32)

        def div_chunk(ci, carry):
            r0 = sid * ROWS_PER_TILE + ci * DCHUNK
            pltpu.sync_copy(acc.at[pl.ds(r0, DCHUNK)], rows_a)

            def row_step(r, c2):
                degv = rows_a[r, pl.ds(D, LANES)]
                dsplat = jnp.full((LANES,), degv[0], jnp.float32)
                scale = 1.0 / jnp.maximum(dsplat, 1.0)
                for j in range(D // LANES):
                    rows_a[r, pl.ds(j * LANES, LANES)] = (
                        rows_a[r, pl.ds(j * LANES, LANES)] * scale)
                rows_a[r, pl.ds(D, LANES)] = e0
                return c2

            lax.fori_loop(0, DCHUNK, row_step, 0)
            pltpu.sync_copy(rows_a, out_hbm.at[pl.ds(r0, DCHUNK)])
            return carry

        lax.fori_loop(0, ROWS_PER_TILE // DCHUNK, div_chunk, 0)

    @pl.when(cid == 0)
    def _():
        run(gidx0, sidx0, tab0, out0)

    @pl.when(cid == 1)
    def _():
        run(gidx1, sidx1, tab1, out1)


@functools.lru_cache(maxsize=None)
def _layer_call():
    return pl.kernel(
        _layer_body,
        out_type=(
            jax.ShapeDtypeStruct((NPAD, W), jnp.float32),
            jax.ShapeDtypeStruct((NPAD, W), jnp.float32),
        ),
        mesh=_mesh(),
        compiler_params=pltpu.CompilerParams(use_tc_tiling_on_sc=False),
        scratch_types=[
            pltpu.VMEM((ECHUNK, W), jnp.float32),
            pltpu.VMEM((ECHUNK, W), jnp.float32),
            pltpu.VMEM((EDGES_PER_TILE,), jnp.int32),
            pltpu.VMEM((EDGES_PER_TILE,), jnp.int32),
            pltpu.VMEM_SHARED((NPAD, W), jnp.float32),
            pltpu.SemaphoreType.DMA,
            pltpu.SemaphoreType.DMA,
        ],
    )


def _layer(tu, ti, user_tab, item_tab, zrows):
    new_item, new_user = _layer_call()(tu, ti, user_tab, ti, tu, item_tab, zrows)
    return new_item, new_user


def _score_body(s16, pu, pi, nu, ni, out_pos, out_neg,
                uv, iv, fv, lv, rows_v, out_v, sem):
    cid = lax.axis_index("c")
    sid = lax.axis_index("s")

    def run(uidx, iidx, out_hbm):
        def step(it, carry):
            off = sid * SCORE_EPT + it * SCHUNK
            pltpu.sync_copy(uidx.at[pl.ds(off, SCHUNK)], uv)
            pltpu.sync_copy(iidx.at[pl.ds(off, SCHUNK)], iv)

            def flat_step(j, c2):
                u = uv[pl.ds(j * LANES, LANES)]
                i = iv[pl.ds(j * LANES, LANES)]
                fv[pl.ds(j * LANES, LANES)] = u * (NPAD // LANES) + (
                    i >> 4)
                lv[pl.ds(j * LANES, LANES)] = i & (LANES - 1)
                return c2

            lax.fori_loop(0, SCHUNK // LANES, flat_step, 0)
            pltpu.async_copy(s16.at[fv], rows_v, sem).wait()
            lane = lax.iota(jnp.int32, LANES)

            def ext_step(j, c2):
                rows_idx = lane + j * LANES
                lanes_idx = lv[pl.ds(j * LANES, LANES)]
                out_v[pl.ds(j * LANES, LANES)] = plsc.load_gather(
                    rows_v, [rows_idx, lanes_idx])
                return c2

            lax.fori_loop(0, SCHUNK // LANES, ext_step, 0)
            pltpu.sync_copy(out_v, out_hbm.at[pl.ds(off, SCHUNK)])
            return carry

        lax.fori_loop(0, SCORE_EPT // SCHUNK, step, 0)

    @pl.when(cid == 0)
    def _():
        run(pu, pi, out_pos)

    @pl.when(cid == 1)
    def _():
        run(nu, ni, out_neg)


@functools.lru_cache(maxsize=None)
def _score_call():
    return pl.kernel(
        _score_body,
        out_type=(
            jax.ShapeDtypeStruct((E,), jnp.float32),
            jax.ShapeDtypeStruct((E,), jnp.float32),
        ),
        mesh=_mesh(),
        compiler_params=pltpu.CompilerParams(
            use_tc_tiling_on_sc=False, needs_layout_passes=False),
        scratch_types=[
            pltpu.VMEM((SCHUNK,), jnp.int32),
            pltpu.VMEM((SCHUNK,), jnp.int32),
            pltpu.VMEM((SCHUNK,), jnp.int32),
            pltpu.VMEM((SCHUNK,), jnp.int32),
            pltpu.VMEM((SCHUNK, LANES), jnp.float32),
            pltpu.VMEM((SCHUNK,), jnp.float32),
            pltpu.SemaphoreType.DMA,
        ],
    )


def _mm_body(x_ref, y_ref, o_ref):
    o_ref[...] = lax.dot_general(
        x_ref[...], y_ref[...],
        dimension_numbers=(((1,), (0,)), ((), ())),
        preferred_element_type=jnp.float32,
    )


def _score_matrix(us2, it2t):
    BM = BN = 1024
    return pl.pallas_call(
        _mm_body,
        grid=(NPAD // BM, NPAD // BN),
        in_specs=[
            pl.BlockSpec((BM, D), lambda i, j: (i, 0)),
            pl.BlockSpec((D, BN), lambda i, j: (0, j)),
        ],
        out_specs=pl.BlockSpec((BM, BN), lambda i, j: (i, j)),
        out_shape=jax.ShapeDtypeStruct((NPAD, NPAD), jnp.float32),
    )(us2, it2t)


def _augment(emb):
    tab = jnp.zeros((NPAD, W), jnp.float32)
    tab = tab.at[:emb.shape[0], :D].set(emb)
    tab = tab.at[:, D].set(1.0)
    return tab


def kernel(train_edge_index, pos_edge_index, neg_edge_index,
           user_embedding, item_embedding):
    tu = train_edge_index[0]
    ti = train_edge_index[1]
    user_tab = _augment(user_embedding)
    item_tab = _augment(item_embedding)
    zrows = jnp.zeros((ROWS_PER_TILE, W), jnp.float32)

    it1, us1 = _layer(tu, ti, user_tab, item_tab, zrows)
    it2, us2 = _layer(tu, ti, us1, it1, zrows)

    s = _score_matrix(us2, jnp.transpose(it2[:, :D]))
    score_pos, score_neg = _score_call()(
        s.reshape(NPAD * NPAD // LANES, LANES),
        pos_edge_index[0], pos_edge_index[1],
        neg_edge_index[0], neg_edge_index[1],
    )
    return (score_pos.reshape(E, 1), score_neg.reshape(E, 1))


# speedup vs baseline: 1.3703x; 1.0454x over previous
import functools

import jax
import jax.numpy as jnp
from jax import lax
from jax.experimental import pallas as pl
from jax.experimental.pallas import tpu as pltpu
from jax.experimental.pallas import tpu_sc as plsc

N_USERS = 5000
N_ITEMS = 5000
D = 256
E = 160000

NC = 2
NS = 16
LANES = 16

NPAD = 5120
W = D + LANES

EDGES_PER_TILE = E // NS
ECHUNK = 40
NSTEPS = EDGES_PER_TILE // ECHUNK
ROWS_PER_TILE = NPAD // NS
DCHUNK = 40

SCORE_EPT = E // NS
SCHUNK = 2000

@functools.lru_cache(maxsize=None)
def _mesh():
    return plsc.VectorSubcoreMesh(
        core_axis_name="c", subcore_axis_name="s", num_cores=NC, num_subcores=NS
    )


def _layer_body(gidx0, sidx0, tab0, gidx1, sidx1, tab1, zf, zd,
                out0, out1, rows_a, rows_b, gidx_v, sidx_v, degbuf,
                ones_v, acc, accd, sem_a, sem_b):
    cid = lax.axis_index("c")
    sid = lax.axis_index("s")

    pltpu.sync_copy(zf, acc.at[pl.ds(sid * ROWS_PER_TILE, ROWS_PER_TILE)])
    pltpu.sync_copy(zd, accd.at[pl.ds(sid * ROWS_PER_TILE, ROWS_PER_TILE)])
    ones16 = jnp.full((LANES,), 1.0, jnp.float32)

    def ones_fill(r, carry):
        ones_v[r] = ones16
        return carry

    lax.fori_loop(0, ECHUNK, ones_fill, 0)
    plsc.subcore_barrier()

    def run(gidx_hbm, sidx_hbm, tab_hbm, out_hbm):
        ebase = sid * EDGES_PER_TILE
        pltpu.sync_copy(gidx_hbm.at[pl.ds(ebase, EDGES_PER_TILE)], gidx_v)
        pltpu.sync_copy(sidx_hbm.at[pl.ds(ebase, EDGES_PER_TILE)], sidx_v)

        def start(s, buf, sem):
            return pltpu.async_copy(
                tab_hbm.at[gidx_v.at[pl.ds(s * ECHUNK, ECHUNK)]], buf, sem)

        def drain(s, buf, sem):
            pltpu.make_async_copy(
                tab_hbm.at[gidx_v.at[pl.ds(0, ECHUNK)]], buf, sem).wait()
            pltpu.sync_copy(
                buf, acc.at[sidx_v.at[pl.ds(s * ECHUNK, ECHUNK)]], add=True)
            pltpu.sync_copy(
                ones_v, accd.at[sidx_v.at[pl.ds(s * ECHUNK, ECHUNK)]],
                add=True)

        start(0, rows_a, sem_a)
        start(1, rows_b, sem_b)

        def edge_step(g, carry):
            s = g * 2
            drain(s, rows_a, sem_a)

            @pl.when(s + 2 < NSTEPS)
            def _():
                start(s + 2, rows_a, sem_a)

            drain(s + 1, rows_b, sem_b)

            @pl.when(s + 3 < NSTEPS)
            def _():
                start(s + 3, rows_b, sem_b)

            return carry

        lax.fori_loop(0, NSTEPS // 2, edge_step, 0)
        plsc.subcore_barrier()

        def div_chunk(ci, carry):
            r0 = sid * ROWS_PER_TILE + ci * DCHUNK
            pltpu.sync_copy(acc.at[pl.ds(r0, DCHUNK)], rows_a)
            pltpu.sync_copy(accd.at[pl.ds(r0, DCHUNK)], degbuf)

            def row_step(r, c2):
                degv = degbuf[r]
                scale = 1.0 / jnp.maximum(degv, 1.0)
                for j in range(D // LANES):
                    rows_a[r, pl.ds(j * LANES, LANES)] = (
                        rows_a[r, pl.ds(j * LANES, LANES)] * scale)
                return c2

            lax.fori_loop(0, DCHUNK, row_step, 0)
            pltpu.sync_copy(rows_a, out_hbm.at[pl.ds(r0, DCHUNK)])
            return carry

        lax.fori_loop(0, ROWS_PER_TILE // DCHUNK, div_chunk, 0)

    @pl.when(cid == 0)
    def _():
        run(gidx0, sidx0, tab0, out0)

    @pl.when(cid == 1)
    def _():
        run(gidx1, sidx1, tab1, out1)


@functools.lru_cache(maxsize=None)
def _layer_call():
    return pl.kernel(
        _layer_body,
        out_type=(
            jax.ShapeDtypeStruct((NPAD, D), jnp.float32),
            jax.ShapeDtypeStruct((NPAD, D), jnp.float32),
        ),
        mesh=_mesh(),
        compiler_params=pltpu.CompilerParams(use_tc_tiling_on_sc=False),
        scratch_types=[
            pltpu.VMEM((ECHUNK, D), jnp.float32),
            pltpu.VMEM((ECHUNK, D), jnp.float32),
            pltpu.VMEM((EDGES_PER_TILE,), jnp.int32),
            pltpu.VMEM((EDGES_PER_TILE,), jnp.int32),
            pltpu.VMEM((DCHUNK, LANES), jnp.float32),
            pltpu.VMEM((ECHUNK, LANES), jnp.float32),
            pltpu.VMEM_SHARED((NPAD, D), jnp.float32),
            pltpu.VMEM_SHARED((NPAD, LANES), jnp.float32),
            pltpu.SemaphoreType.DMA,
            pltpu.SemaphoreType.DMA,
        ],
    )


def _layer(tu, ti, user_tab, item_tab, zf, zd):
    new_item, new_user = _layer_call()(
        tu, ti, user_tab, ti, tu, item_tab, zf, zd)
    return new_item, new_user


def _score_body(s16, pu, pi, nu, ni, out_pos, out_neg,
                uv, iv, fv, lv, rows_v, out_v, sem):
    cid = lax.axis_index("c")
    sid = lax.axis_index("s")

    def run(uidx, iidx, out_hbm):
        def step(it, carry):
            off = sid * SCORE_EPT + it * SCHUNK
            pltpu.sync_copy(uidx.at[pl.ds(off, SCHUNK)], uv)
            pltpu.sync_copy(iidx.at[pl.ds(off, SCHUNK)], iv)

            def flat_step(j, c2):
                u = uv[pl.ds(j * LANES, LANES)]
                i = iv[pl.ds(j * LANES, LANES)]
                fv[pl.ds(j * LANES, LANES)] = u * (NPAD // LANES) + (
                    i >> 4)
                lv[pl.ds(j * LANES, LANES)] = i & (LANES - 1)
                return c2

            lax.fori_loop(0, SCHUNK // LANES, flat_step, 0)
            pltpu.async_copy(s16.at[fv], rows_v, sem).wait()
            lane = lax.iota(jnp.int32, LANES)

            def ext_step(j, c2):
                rows_idx = lane + j * LANES
                lanes_idx = lv[pl.ds(j * LANES, LANES)]
                out_v[pl.ds(j * LANES, LANES)] = plsc.load_gather(
                    rows_v, [rows_idx, lanes_idx])
                return c2

            lax.fori_loop(0, SCHUNK // LANES, ext_step, 0)
            pltpu.sync_copy(out_v, out_hbm.at[pl.ds(off, SCHUNK)])
            return carry

        lax.fori_loop(0, SCORE_EPT // SCHUNK, step, 0)

    @pl.when(cid == 0)
    def _():
        run(pu, pi, out_pos)

    @pl.when(cid == 1)
    def _():
        run(nu, ni, out_neg)


@functools.lru_cache(maxsize=None)
def _score_call():
    return pl.kernel(
        _score_body,
        out_type=(
            jax.ShapeDtypeStruct((E,), jnp.float32),
            jax.ShapeDtypeStruct((E,), jnp.float32),
        ),
        mesh=_mesh(),
        compiler_params=pltpu.CompilerParams(
            use_tc_tiling_on_sc=False, needs_layout_passes=False),
        scratch_types=[
            pltpu.VMEM((SCHUNK,), jnp.int32),
            pltpu.VMEM((SCHUNK,), jnp.int32),
            pltpu.VMEM((SCHUNK,), jnp.int32),
            pltpu.VMEM((SCHUNK,), jnp.int32),
            pltpu.VMEM((SCHUNK, LANES), jnp.float32),
            pltpu.VMEM((SCHUNK,), jnp.float32),
            pltpu.SemaphoreType.DMA,
        ],
    )


def _mm_body(x_ref, y_ref, o_ref):
    o_ref[...] = lax.dot_general(
        x_ref[...], y_ref[...],
        dimension_numbers=(((1,), (0,)), ((), ())),
        preferred_element_type=jnp.float32,
    )


def _score_matrix(us2, it2t):
    BM = BN = 1024
    return pl.pallas_call(
        _mm_body,
        grid=(NPAD // BM, NPAD // BN),
        in_specs=[
            pl.BlockSpec((BM, D), lambda i, j: (i, 0)),
            pl.BlockSpec((D, BN), lambda i, j: (0, j)),
        ],
        out_specs=pl.BlockSpec((BM, BN), lambda i, j: (i, j)),
        out_shape=jax.ShapeDtypeStruct((NPAD, NPAD), jnp.float32),
    )(us2, it2t)


def kernel(train_edge_index, pos_edge_index, neg_edge_index,
           user_embedding, item_embedding):
    tu = train_edge_index[0]
    ti = train_edge_index[1]
    zf = jnp.zeros((ROWS_PER_TILE, D), jnp.float32)
    zd = jnp.zeros((ROWS_PER_TILE, LANES), jnp.float32)

    it1, us1 = _layer(tu, ti, user_embedding, item_embedding, zf, zd)
    it2, us2 = _layer(tu, ti, us1, it1, zf, zd)

    s = _score_matrix(us2, jnp.transpose(it2))
    score_pos, score_neg = _score_call()(
        s.reshape(NPAD * NPAD // LANES, LANES),
        pos_edge_index[0], pos_edge_index[1],
        neg_edge_index[0], neg_edge_index[1],
    )
    return (score_pos.reshape(E, 1), score_neg.reshape(E, 1))


# speedup vs baseline: 1.3754x; 1.0038x over previous
import functools

import jax
import jax.numpy as jnp
from jax import lax
from jax.experimental import pallas as pl
from jax.experimental.pallas import tpu as pltpu
from jax.experimental.pallas import tpu_sc as plsc

N_USERS = 5000
N_ITEMS = 5000
D = 256
E = 160000

NC = 2
NS = 16
LANES = 16

NPAD = 5120
W = D + LANES

EDGES_PER_TILE = E // NS
ECHUNK = 40
NSTEPS = EDGES_PER_TILE // ECHUNK
ROWS_PER_TILE = NPAD // NS
DCHUNK = 40

SCORE_EPT = E // NS
SCHUNK = 2000

@functools.lru_cache(maxsize=None)
def _mesh():
    return plsc.VectorSubcoreMesh(
        core_axis_name="c", subcore_axis_name="s", num_cores=NC, num_subcores=NS
    )


def _layer_body(gidx0, sidx0, tab0, gidx1, sidx1, tab1, zf, zd,
                out0, out1, rows_a, rows_b, gidx_v, sidx_v, degbuf,
                ones_v, acc, accd, sem_a, sem_b, sem_c, sem_d):
    cid = lax.axis_index("c")
    sid = lax.axis_index("s")

    pltpu.sync_copy(zf, acc.at[pl.ds(sid * ROWS_PER_TILE, ROWS_PER_TILE)])
    pltpu.sync_copy(zd, accd.at[pl.ds(sid * ROWS_PER_TILE, ROWS_PER_TILE)])
    ones16 = jnp.full((LANES,), 1.0, jnp.float32)

    def ones_fill(r, carry):
        ones_v[r] = ones16
        return carry

    lax.fori_loop(0, ECHUNK, ones_fill, 0)
    plsc.subcore_barrier()

    def run(gidx_hbm, sidx_hbm, tab_hbm, out_hbm):
        ebase = sid * EDGES_PER_TILE
        pltpu.sync_copy(gidx_hbm.at[pl.ds(ebase, EDGES_PER_TILE)], gidx_v)
        pltpu.sync_copy(sidx_hbm.at[pl.ds(ebase, EDGES_PER_TILE)], sidx_v)

        def start(s, buf, sem):
            return pltpu.async_copy(
                tab_hbm.at[gidx_v.at[pl.ds(s * ECHUNK, ECHUNK)]], buf, sem)

        def drain(s, buf, sem):
            pltpu.make_async_copy(
                tab_hbm.at[gidx_v.at[pl.ds(0, ECHUNK)]], buf, sem).wait()
            pltpu.sync_copy(
                buf, acc.at[sidx_v.at[pl.ds(s * ECHUNK, ECHUNK)]], add=True)
            pltpu.sync_copy(
                ones_v, accd.at[sidx_v.at[pl.ds(s * ECHUNK, ECHUNK)]],
                add=True)

        start(0, rows_a, sem_a)
        start(1, rows_b, sem_b)

        def edge_step(g, carry):
            s = g * 2
            drain(s, rows_a, sem_a)

            @pl.when(s + 2 < NSTEPS)
            def _():
                start(s + 2, rows_a, sem_a)

            drain(s + 1, rows_b, sem_b)

            @pl.when(s + 3 < NSTEPS)
            def _():
                start(s + 3, rows_b, sem_b)

            return carry

        lax.fori_loop(0, NSTEPS // 2, edge_step, 0)
        plsc.subcore_barrier()

        NCH = ROWS_PER_TILE // DCHUNK
        bufs = (rows_a, rows_b)
        sem_in = (sem_a, sem_b)
        sem_out = (sem_c, sem_d)

        def rbase(ci):
            return sid * ROWS_PER_TILE + ci * DCHUNK

        in_d = {0: pltpu.async_copy(
            acc.at[pl.ds(rbase(0), DCHUNK)], bufs[0], sem_in[0])}
        out_d = {}
        for ci in range(NCH):
            b = ci % 2
            if ci + 1 < NCH:
                if ci - 1 >= 0:
                    out_d[ci - 1].wait()
                in_d[ci + 1] = pltpu.async_copy(
                    acc.at[pl.ds(rbase(ci + 1), DCHUNK)],
                    bufs[1 - b], sem_in[1 - b])
            in_d[ci].wait()
            pltpu.sync_copy(accd.at[pl.ds(rbase(ci), DCHUNK)], degbuf)

            def row_step(r, c2, _buf=bufs[b]):
                degv = degbuf[r]
                scale = 1.0 / jnp.maximum(degv, 1.0)
                for j in range(D // LANES):
                    _buf[r, pl.ds(j * LANES, LANES)] = (
                        _buf[r, pl.ds(j * LANES, LANES)] * scale)
                return c2

            lax.fori_loop(0, DCHUNK, row_step, 0)
            out_d[ci] = pltpu.async_copy(
                bufs[b], out_hbm.at[pl.ds(rbase(ci), DCHUNK)], sem_out[b])
        out_d[NCH - 2].wait()
        out_d[NCH - 1].wait()

    @pl.when(cid == 0)
    def _():
        run(gidx0, sidx0, tab0, out0)

    @pl.when(cid == 1)
    def _():
        run(gidx1, sidx1, tab1, out1)


@functools.lru_cache(maxsize=None)
def _layer_call():
    return pl.kernel(
        _layer_body,
        out_type=(
            jax.ShapeDtypeStruct((NPAD, D), jnp.float32),
            jax.ShapeDtypeStruct((NPAD, D), jnp.float32),
        ),
        mesh=_mesh(),
        compiler_params=pltpu.CompilerParams(use_tc_tiling_on_sc=False),
        scratch_types=[
            pltpu.VMEM((ECHUNK, D), jnp.float32),
            pltpu.VMEM((ECHUNK, D), jnp.float32),
            pltpu.VMEM((EDGES_PER_TILE,), jnp.int32),
            pltpu.VMEM((EDGES_PER_TILE,), jnp.int32),
            pltpu.VMEM((DCHUNK, LANES), jnp.float32),
            pltpu.VMEM((ECHUNK, LANES), jnp.float32),
            pltpu.VMEM_SHARED((NPAD, D), jnp.float32),
            pltpu.VMEM_SHARED((NPAD, LANES), jnp.float32),
            pltpu.SemaphoreType.DMA,
            pltpu.SemaphoreType.DMA,
            pltpu.SemaphoreType.DMA,
            pltpu.SemaphoreType.DMA,
        ],
    )


def _layer(tu, ti, user_tab, item_tab, zf, zd):
    new_item, new_user = _layer_call()(
        tu, ti, user_tab, ti, tu, item_tab, zf, zd)
    return new_item, new_user


def _score_body(s16, pu, pi, nu, ni, out_pos, out_neg,
                uv, iv, fv, lv, rows_v, out_v, sem):
    cid = lax.axis_index("c")
    sid = lax.axis_index("s")

    def run(uidx, iidx, out_hbm):
        def step(it, carry):
            off = sid * SCORE_EPT + it * SCHUNK
            pltpu.sync_copy(uidx.at[pl.ds(off, SCHUNK)], uv)
            pltpu.sync_copy(iidx.at[pl.ds(off, SCHUNK)], iv)

            def flat_step(j, c2):
                u = uv[pl.ds(j * LANES, LANES)]
                i = iv[pl.ds(j * LANES, LANES)]
                fv[pl.ds(j * LANES, LANES)] = u * (NPAD // LANES) + (
                    i >> 4)
                lv[pl.ds(j * LANES, LANES)] = i & (LANES - 1)
                return c2

            lax.fori_loop(0, SCHUNK // LANES, flat_step, 0)
            pltpu.async_copy(s16.at[fv], rows_v, sem).wait()
            lane = lax.iota(jnp.int32, LANES)

            def ext_step(j, c2):
                rows_idx = lane + j * LANES
                lanes_idx = lv[pl.ds(j * LANES, LANES)]
                out_v[pl.ds(j * LANES, LANES)] = plsc.load_gather(
                    rows_v, [rows_idx, lanes_idx])
                return c2

            lax.fori_loop(0, SCHUNK // LANES, ext_step, 0)
            pltpu.sync_copy(out_v, out_hbm.at[pl.ds(off, SCHUNK)])
            return carry

        lax.fori_loop(0, SCORE_EPT // SCHUNK, step, 0)

    @pl.when(cid == 0)
    def _():
        run(pu, pi, out_pos)

    @pl.when(cid == 1)
    def _():
        run(nu, ni, out_neg)


@functools.lru_cache(maxsize=None)
def _score_call():
    return pl.kernel(
        _score_body,
        out_type=(
            jax.ShapeDtypeStruct((E,), jnp.float32),
            jax.ShapeDtypeStruct((E,), jnp.float32),
        ),
        mesh=_mesh(),
        compiler_params=pltpu.CompilerParams(
            use_tc_tiling_on_sc=False, needs_layout_passes=False),
        scratch_types=[
            pltpu.VMEM((SCHUNK,), jnp.int32),
            pltpu.VMEM((SCHUNK,), jnp.int32),
            pltpu.VMEM((SCHUNK,), jnp.int32),
            pltpu.VMEM((SCHUNK,), jnp.int32),
            pltpu.VMEM((SCHUNK, LANES), jnp.float32),
            pltpu.VMEM((SCHUNK,), jnp.float32),
            pltpu.SemaphoreType.DMA,
        ],
    )


def _mm_body(x_ref, y_ref, o_ref):
    o_ref[...] = lax.dot_general(
        x_ref[...], y_ref[...],
        dimension_numbers=(((1,), (0,)), ((), ())),
        preferred_element_type=jnp.float32,
    )


def _score_matrix(us2, it2t):
    BM = BN = 1024
    return pl.pallas_call(
        _mm_body,
        grid=(NPAD // BM, NPAD // BN),
        in_specs=[
            pl.BlockSpec((BM, D), lambda i, j: (i, 0)),
            pl.BlockSpec((D, BN), lambda i, j: (0, j)),
        ],
        out_specs=pl.BlockSpec((BM, BN), lambda i, j: (i, j)),
        out_shape=jax.ShapeDtypeStruct((NPAD, NPAD), jnp.float32),
    )(us2, it2t)


def kernel(train_edge_index, pos_edge_index, neg_edge_index,
           user_embedding, item_embedding):
    tu = train_edge_index[0]
    ti = train_edge_index[1]
    zf = jnp.zeros((ROWS_PER_TILE, D), jnp.float32)
    zd = jnp.zeros((ROWS_PER_TILE, LANES), jnp.float32)

    it1, us1 = _layer(tu, ti, user_embedding, item_embedding, zf, zd)
    it2, us2 = _layer(tu, ti, us1, it1, zf, zd)

    s = _score_matrix(us2, jnp.transpose(it2))
    score_pos, score_neg = _score_call()(
        s.reshape(NPAD * NPAD // LANES, LANES),
        pos_edge_index[0], pos_edge_index[1],
        neg_edge_index[0], neg_edge_index[1],
    )
    return (score_pos.reshape(E, 1), score_neg.reshape(E, 1))
